# baseline (speedup 1.0000x reference)
import functools

import jax
import jax.numpy as jnp
from jax import lax
from jax.experimental import pallas as pl
from jax.experimental.pallas import tpu as pltpu
from jax.experimental.pallas import tpu_sc as plsc

CHUNK = 512
NBUF = 2


def _gather_kernel(n_rows, d, n_workers, num_cores):
    b_per_w = n_rows // n_workers
    nchunks = b_per_w // CHUNK
    ngroups = nchunks // NBUF
    mesh = plsc.VectorSubcoreMesh(core_axis_name="c", subcore_axis_name="s")

    @functools.partial(
        pl.kernel,
        mesh=mesh,
        compiler_params=pltpu.CompilerParams(use_tc_tiling_on_sc=False),
        out_type=jax.ShapeDtypeStruct((n_rows, d), jnp.float32),
        scratch_types=[
            pltpu.VMEM((b_per_w,), jnp.int32),
            pltpu.VMEM((NBUF, CHUNK, d), jnp.float32),
            pltpu.SemaphoreType.DMA,
            pltpu.SemaphoreType.DMA,
        ],
    )
    def k(ids_hbm, tab_hbm, out_hbm, idx_v, rows_v, sem0, sem1):
        wid = lax.axis_index("s") * num_cores + lax.axis_index("c")
        base = wid * b_per_w
        sems = (sem0, sem1)
        pltpu.sync_copy(ids_hbm.at[pl.ds(base, b_per_w)], idx_v)

        def start(j, b):
            pltpu.async_copy(
                tab_hbm.at[idx_v.at[pl.ds(j * CHUNK, CHUNK)]],
                rows_v.at[b],
                sems[b],
            )

        def wait(b):
            pltpu.make_async_copy(
                tab_hbm.at[idx_v.at[pl.ds(0, CHUNK)]],
                rows_v.at[b],
                sems[b],
            ).wait()

        for b in range(NBUF):
            start(b, b)

        def body(g, carry):
            for b in range(NBUF):
                j = g * NBUF + b
                wait(b)
                pltpu.sync_copy(
                    rows_v.at[b], out_hbm.at[pl.ds(base + j * CHUNK, CHUNK)]
                )
                jn = j + NBUF

                @pl.when(jn < nchunks)
                def _():
                    start(jn, b)

            return carry

        lax.fori_loop(0, ngroups, body, 0)

    return k


def kernel(token_ids, embed_weight):
    bt, s = token_ids.shape
    v, d = embed_weight.shape
    n = bt * s
    flat_ids = token_ids.reshape(n).astype(jnp.int32)
    info = plsc.get_sparse_core_info()
    n_workers = info.num_cores * info.num_subcores
    out = _gather_kernel(n, d, n_workers, info.num_cores)(flat_ids, embed_weight)
    return out.reshape(bt, s, d)
